# baseline (device time: 62009 ns/iter reference)
import jax
import jax.numpy as jnp
from jax import lax
from jax.experimental import pallas as pl
from jax.experimental.pallas import tpu as pltpu

N_Z = 4
K = 8


def _allreduce_z(partial):
    t, d = partial.shape
    rows = t // K

    def body(p_ref, out_ref, lbuf, rbuf, sbuf_r, sbuf_l,
             r_send, r_recv, l_send, l_recv):
        my_x = lax.axis_index("x")
        my_y = lax.axis_index("y")
        my_z = lax.axis_index("z")
        has_left = my_z > 0
        has_right = my_z < N_Z - 1
        is_middle = jnp.logical_and(has_left, has_right)
        left = lax.rem(my_z - 1 + N_Z, N_Z)
        right = lax.rem(my_z + 1, N_Z)

        barrier_sem = pltpu.get_barrier_semaphore()

        @pl.when(has_left)
        def _():
            pl.semaphore_signal(
                barrier_sem, inc=1,
                device_id=(my_x, my_y, left),
                device_id_type=pl.DeviceIdType.MESH,
            )

        @pl.when(has_right)
        def _():
            pl.semaphore_signal(
                barrier_sem, inc=1,
                device_id=(my_x, my_y, right),
                device_id_type=pl.DeviceIdType.MESH,
            )

        pl.semaphore_wait(barrier_sem, 1)

        @pl.when(is_middle)
        def _():
            pl.semaphore_wait(barrier_sem, 1)

        r_rdmas = []
        l_rdmas = []
        r_owns = []
        l_owns = []
        for k in range(K):
            ro = pl.ds(k * rows, rows)
            r_rdmas.append(pltpu.make_async_remote_copy(
                src_ref=sbuf_r.at[k],
                dst_ref=lbuf.at[k],
                send_sem=r_send.at[k],
                recv_sem=r_recv.at[k],
                device_id=(my_x, my_y, right),
                device_id_type=pl.DeviceIdType.MESH,
            ))
            r_owns.append(pltpu.make_async_remote_copy(
                src_ref=p_ref.at[ro, :],
                dst_ref=lbuf.at[k],
                send_sem=r_send.at[k],
                recv_sem=r_recv.at[k],
                device_id=(my_x, my_y, right),
                device_id_type=pl.DeviceIdType.MESH,
            ))
            l_rdmas.append(pltpu.make_async_remote_copy(
                src_ref=sbuf_l.at[k],
                dst_ref=rbuf.at[k],
                send_sem=l_send.at[k],
                recv_sem=l_recv.at[k],
                device_id=(my_x, my_y, left),
                device_id_type=pl.DeviceIdType.MESH,
            ))
            l_owns.append(pltpu.make_async_remote_copy(
                src_ref=p_ref.at[ro, :],
                dst_ref=rbuf.at[k],
                send_sem=l_send.at[k],
                recv_sem=l_recv.at[k],
                device_id=(my_x, my_y, left),
                device_id_type=pl.DeviceIdType.MESH,
            ))

        for k in range(K):
            @pl.when(my_z == 0)
            def _():
                r_owns[k].start()

            @pl.when(my_z == N_Z - 1)
            def _():
                l_owns[k].start()

        for k in range(K):
            ro = pl.ds(k * rows, rows)

            @pl.when(has_left)
            def _():
                r_rdmas[k].wait_recv()

            @pl.when(is_middle)
            def _():
                sbuf_r[k, :, :] = lbuf[k, :, :] + p_ref[ro, :]
                r_rdmas[k].start()

            @pl.when(has_right)
            def _():
                l_rdmas[k].wait_recv()

            @pl.when(is_middle)
            def _():
                sbuf_l[k, :, :] = rbuf[k, :, :] + p_ref[ro, :]
                l_rdmas[k].start()

            @pl.when(is_middle)
            def _():
                out_ref[ro, :] = sbuf_r[k, :, :] + rbuf[k, :, :]

            @pl.when(my_z == 0)
            def _():
                out_ref[ro, :] = p_ref[ro, :] + rbuf[k, :, :]

            @pl.when(my_z == N_Z - 1)
            def _():
                out_ref[ro, :] = p_ref[ro, :] + lbuf[k, :, :]

        for k in range(K):
            @pl.when(has_right)
            def _():
                r_rdmas[k].wait_send()

            @pl.when(has_left)
            def _():
                l_rdmas[k].wait_send()

    return pl.pallas_call(
        body,
        out_shape=jax.ShapeDtypeStruct((t, d), partial.dtype),
        in_specs=[pl.BlockSpec(memory_space=pltpu.VMEM)],
        out_specs=pl.BlockSpec(memory_space=pltpu.VMEM),
        scratch_shapes=[
            pltpu.VMEM((K, rows, d), partial.dtype),
            pltpu.VMEM((K, rows, d), partial.dtype),
            pltpu.VMEM((K, rows, d), partial.dtype),
            pltpu.VMEM((K, rows, d), partial.dtype),
            pltpu.SemaphoreType.DMA((K,)),
            pltpu.SemaphoreType.DMA((K,)),
            pltpu.SemaphoreType.DMA((K,)),
            pltpu.SemaphoreType.DMA((K,)),
        ],
        compiler_params=pltpu.CompilerParams(collective_id=0),
    )(partial)


def kernel(ids, E):
    v_per, _ = E.shape
    z = lax.axis_index("z")
    local = ids - z * v_per
    mask = (local >= 0) & (local < v_per)
    safe = jnp.where(mask, local, 0)
    partial = jnp.where(mask[:, None], jnp.take(E, safe, axis=0), 0.0)
    return _allreduce_z(partial.astype(jnp.float32))


# device time: 55224 ns/iter; 1.1229x vs baseline; 1.1229x over previous
import jax
import jax.numpy as jnp
from jax import lax
from jax.experimental import pallas as pl
from jax.experimental.pallas import tpu as pltpu

N_Z = 4
K = 8


def _allreduce_z(partial):
    t, d = partial.shape
    rows = t // K

    def body(p_ref, out_ref, lbuf, rbuf, sbuf_r, sbuf_l,
             r_send, r_recv, l_send, l_recv):
        my_x = lax.axis_index("x")
        my_y = lax.axis_index("y")
        my_z = lax.axis_index("z")
        has_left = my_z > 0
        has_right = my_z < N_Z - 1
        is_middle = jnp.logical_and(has_left, has_right)
        left = lax.rem(my_z - 1 + N_Z, N_Z)
        right = lax.rem(my_z + 1, N_Z)

        barrier_sem = pltpu.get_barrier_semaphore()

        @pl.when(has_left)
        def _():
            pl.semaphore_signal(
                barrier_sem, inc=1,
                device_id=(my_x, my_y, left),
                device_id_type=pl.DeviceIdType.MESH,
            )

        @pl.when(has_right)
        def _():
            pl.semaphore_signal(
                barrier_sem, inc=1,
                device_id=(my_x, my_y, right),
                device_id_type=pl.DeviceIdType.MESH,
            )

        pl.semaphore_wait(barrier_sem, 1)

        @pl.when(is_middle)
        def _():
            pl.semaphore_wait(barrier_sem, 1)

        r_rdmas = []
        l_rdmas = []
        r_owns = []
        l_owns = []
        for k in range(K):
            ro = pl.ds(k * rows, rows)
            r_rdmas.append(pltpu.make_async_remote_copy(
                src_ref=sbuf_r.at[k],
                dst_ref=lbuf.at[k],
                send_sem=r_send.at[k],
                recv_sem=r_recv.at[k],
                device_id=(my_x, my_y, right),
                device_id_type=pl.DeviceIdType.MESH,
            ))
            r_owns.append(pltpu.make_async_remote_copy(
                src_ref=p_ref.at[ro, :],
                dst_ref=lbuf.at[k],
                send_sem=r_send.at[k],
                recv_sem=r_recv.at[k],
                device_id=(my_x, my_y, right),
                device_id_type=pl.DeviceIdType.MESH,
            ))
            l_rdmas.append(pltpu.make_async_remote_copy(
                src_ref=sbuf_l.at[k],
                dst_ref=rbuf.at[k],
                send_sem=l_send.at[k],
                recv_sem=l_recv.at[k],
                device_id=(my_x, my_y, left),
                device_id_type=pl.DeviceIdType.MESH,
            ))
            l_owns.append(pltpu.make_async_remote_copy(
                src_ref=p_ref.at[ro, :],
                dst_ref=rbuf.at[k],
                send_sem=l_send.at[k],
                recv_sem=l_recv.at[k],
                device_id=(my_x, my_y, left),
                device_id_type=pl.DeviceIdType.MESH,
            ))

        for k in range(K):
            @pl.when(my_z == 0)
            def _():
                r_owns[k].start()

            @pl.when(my_z == N_Z - 1)
            def _():
                l_owns[k].start()

        OFF = 3
        for k in range(K + OFF):
            if k >= OFF:
                j = k - OFF
                rj = pl.ds(j * rows, rows)

                @pl.when(has_right)
                def _():
                    l_rdmas[j].wait_recv()

                @pl.when(is_middle)
                def _():
                    sbuf_l[j, :, :] = rbuf[j, :, :] + p_ref[rj, :]
                    l_rdmas[j].start()

                @pl.when(is_middle)
                def _():
                    out_ref[rj, :] = sbuf_r[j, :, :] + rbuf[j, :, :]

                @pl.when(my_z == 0)
                def _():
                    out_ref[rj, :] = p_ref[rj, :] + rbuf[j, :, :]

                @pl.when(my_z == N_Z - 1)
                def _():
                    out_ref[rj, :] = p_ref[rj, :] + lbuf[j, :, :]

            if k < K:
                ro = pl.ds(k * rows, rows)

                @pl.when(has_left)
                def _():
                    r_rdmas[k].wait_recv()

                @pl.when(is_middle)
                def _():
                    sbuf_r[k, :, :] = lbuf[k, :, :] + p_ref[ro, :]
                    r_rdmas[k].start()

        for k in range(K):
            @pl.when(has_right)
            def _():
                r_rdmas[k].wait_send()

            @pl.when(has_left)
            def _():
                l_rdmas[k].wait_send()

    return pl.pallas_call(
        body,
        out_shape=jax.ShapeDtypeStruct((t, d), partial.dtype),
        in_specs=[pl.BlockSpec(memory_space=pltpu.VMEM)],
        out_specs=pl.BlockSpec(memory_space=pltpu.VMEM),
        scratch_shapes=[
            pltpu.VMEM((K, rows, d), partial.dtype),
            pltpu.VMEM((K, rows, d), partial.dtype),
            pltpu.VMEM((K, rows, d), partial.dtype),
            pltpu.VMEM((K, rows, d), partial.dtype),
            pltpu.SemaphoreType.DMA((K,)),
            pltpu.SemaphoreType.DMA((K,)),
            pltpu.SemaphoreType.DMA((K,)),
            pltpu.SemaphoreType.DMA((K,)),
        ],
        compiler_params=pltpu.CompilerParams(collective_id=0),
    )(partial)


def kernel(ids, E):
    v_per, _ = E.shape
    z = lax.axis_index("z")
    local = ids - z * v_per
    mask = (local >= 0) & (local < v_per)
    safe = jnp.where(mask, local, 0)
    partial = jnp.where(mask[:, None], jnp.take(E, safe, axis=0), 0.0)
    return _allreduce_z(partial.astype(jnp.float32))


# device time: 28728 ns/iter; 2.1585x vs baseline; 1.9223x over previous
import jax
import jax.numpy as jnp
from jax import lax
from jax.experimental import pallas as pl
from jax.experimental.pallas import tpu as pltpu

N_Z = 4


def _allreduce_z(partial):
    t, d = partial.shape
    g = t // N_Z

    def body(p_ref, out_ref, rs_buf, rs_send, rs_recv, ag_send, ag_recv):
        my_x = lax.axis_index("x")
        my_y = lax.axis_index("y")
        my_z = lax.axis_index("z")

        barrier_sem = pltpu.get_barrier_semaphore()
        for nbr in range(N_Z):
            @pl.when(my_z != nbr)
            def _():
                pl.semaphore_signal(
                    barrier_sem, inc=1,
                    device_id=(my_x, my_y, nbr),
                    device_id_type=pl.DeviceIdType.MESH,
                )
        pl.semaphore_wait(barrier_sem, N_Z - 1)

        rs = {}
        ag = {}
        for s in range(N_Z):
            for tt in range(N_Z):
                if tt == s:
                    continue
                rs[(s, tt)] = pltpu.make_async_remote_copy(
                    src_ref=p_ref.at[pl.ds(tt * g, g), :],
                    dst_ref=rs_buf.at[s],
                    send_sem=rs_send.at[tt],
                    recv_sem=rs_recv.at[s],
                    device_id=(my_x, my_y, tt),
                    device_id_type=pl.DeviceIdType.MESH,
                )
                ag[(s, tt)] = pltpu.make_async_remote_copy(
                    src_ref=out_ref.at[pl.ds(s * g, g), :],
                    dst_ref=out_ref.at[pl.ds(s * g, g), :],
                    send_sem=ag_send.at[tt],
                    recv_sem=ag_recv.at[s],
                    device_id=(my_x, my_y, tt),
                    device_id_type=pl.DeviceIdType.MESH,
                )

        for s in range(N_Z):
            @pl.when(my_z == s)
            def _():
                for tt in range(N_Z):
                    if tt != s:
                        rs[(s, tt)].start()

        for tt in range(N_Z):
            @pl.when(my_z == tt)
            def _():
                for s in range(N_Z):
                    if s != tt:
                        rs[(s, tt)].wait_recv()

        for j in range(N_Z):
            @pl.when(my_z == j)
            def _():
                others = [s for s in range(N_Z) if s != j]
                ro = pl.ds(j * g, g)
                out_ref[ro, :] = (
                    p_ref[ro, :]
                    + rs_buf[others[0], :, :]
                    + rs_buf[others[1], :, :]
                    + rs_buf[others[2], :, :]
                )
                for tt in others:
                    ag[(j, tt)].start()

        for tt in range(N_Z):
            @pl.when(my_z == tt)
            def _():
                for s in range(N_Z):
                    if s != tt:
                        ag[(s, tt)].wait_recv()

        for s in range(N_Z):
            @pl.when(my_z == s)
            def _():
                for tt in range(N_Z):
                    if tt != s:
                        rs[(s, tt)].wait_send()
                        ag[(s, tt)].wait_send()

    return pl.pallas_call(
        body,
        out_shape=jax.ShapeDtypeStruct((t, d), partial.dtype),
        in_specs=[pl.BlockSpec(memory_space=pltpu.VMEM)],
        out_specs=pl.BlockSpec(memory_space=pltpu.VMEM),
        scratch_shapes=[
            pltpu.VMEM((N_Z, g, d), partial.dtype),
            pltpu.SemaphoreType.DMA((N_Z,)),
            pltpu.SemaphoreType.DMA((N_Z,)),
            pltpu.SemaphoreType.DMA((N_Z,)),
            pltpu.SemaphoreType.DMA((N_Z,)),
        ],
        compiler_params=pltpu.CompilerParams(collective_id=0),
    )(partial)


def kernel(ids, E):
    v_per, _ = E.shape
    z = lax.axis_index("z")
    local = ids - z * v_per
    mask = (local >= 0) & (local < v_per)
    safe = jnp.where(mask, local, 0)
    partial = jnp.where(mask[:, None], jnp.take(E, safe, axis=0), 0.0)
    return _allreduce_z(partial.astype(jnp.float32))


# device time: 24845 ns/iter; 2.4958x vs baseline; 1.1563x over previous
import jax
import jax.numpy as jnp
from jax import lax
from jax.experimental import pallas as pl
from jax.experimental.pallas import tpu as pltpu

N_Z = 4


def _allreduce_z(partial):
    t, d = partial.shape
    half = t // 2
    g = half // N_Z

    def body(p_ref, out_ref, rs_buf, rs_send, rs_recv, ag_send, ag_recv,
             x_send, x_recv):
        my_x = lax.axis_index("x")
        my_y = lax.axis_index("y")
        my_z = lax.axis_index("z")
        base = my_x * half
        obase = (1 - my_x) * half

        barrier_sem = pltpu.get_barrier_semaphore()
        for nbr in range(N_Z):
            @pl.when(my_z != nbr)
            def _():
                pl.semaphore_signal(
                    barrier_sem, inc=1,
                    device_id=(my_x, my_y, nbr),
                    device_id_type=pl.DeviceIdType.MESH,
                )
        pl.semaphore_signal(
            barrier_sem, inc=1,
            device_id=(1 - my_x, my_y, my_z),
            device_id_type=pl.DeviceIdType.MESH,
        )
        pl.semaphore_wait(barrier_sem, N_Z)

        rs = {}
        ag = {}
        for s in range(N_Z):
            for tt in range(N_Z):
                if tt == s:
                    continue
                rs[(s, tt)] = pltpu.make_async_remote_copy(
                    src_ref=p_ref.at[pl.ds(base + tt * g, g), :],
                    dst_ref=rs_buf.at[s],
                    send_sem=rs_send.at[tt],
                    recv_sem=rs_recv.at[s],
                    device_id=(my_x, my_y, tt),
                    device_id_type=pl.DeviceIdType.MESH,
                )
                ag[(s, tt)] = pltpu.make_async_remote_copy(
                    src_ref=out_ref.at[pl.ds(base + s * g, g), :],
                    dst_ref=out_ref.at[pl.ds(base + s * g, g), :],
                    send_sem=ag_send.at[tt],
                    recv_sem=ag_recv.at[s],
                    device_id=(my_x, my_y, tt),
                    device_id_type=pl.DeviceIdType.MESH,
                )
        x_snd = []
        x_rcv = []
        for s in range(N_Z):
            x_snd.append(pltpu.make_async_remote_copy(
                src_ref=out_ref.at[pl.ds(base + s * g, g), :],
                dst_ref=out_ref.at[pl.ds(base + s * g, g), :],
                send_sem=x_send.at[s],
                recv_sem=x_recv.at[s],
                device_id=(1 - my_x, my_y, my_z),
                device_id_type=pl.DeviceIdType.MESH,
            ))
            x_rcv.append(pltpu.make_async_remote_copy(
                src_ref=out_ref.at[pl.ds(obase + s * g, g), :],
                dst_ref=out_ref.at[pl.ds(obase + s * g, g), :],
                send_sem=x_send.at[s],
                recv_sem=x_recv.at[s],
                device_id=(1 - my_x, my_y, my_z),
                device_id_type=pl.DeviceIdType.MESH,
            ))

        for s in range(N_Z):
            @pl.when(my_z == s)
            def _():
                for tt in range(N_Z):
                    if tt != s:
                        rs[(s, tt)].start()

        for tt in range(N_Z):
            @pl.when(my_z == tt)
            def _():
                for s in range(N_Z):
                    if s != tt:
                        rs[(s, tt)].wait_recv()

        for j in range(N_Z):
            @pl.when(my_z == j)
            def _():
                others = [s for s in range(N_Z) if s != j]
                ro = pl.ds(base + j * g, g)
                out_ref[ro, :] = (
                    p_ref[ro, :]
                    + rs_buf[others[0], :, :]
                    + rs_buf[others[1], :, :]
                    + rs_buf[others[2], :, :]
                )
                for tt in others:
                    ag[(j, tt)].start()
                x_snd[j].start()

        for tt in range(N_Z):
            @pl.when(my_z == tt)
            def _():
                for s in range(N_Z):
                    if s != tt:
                        ag[(s, tt)].wait_recv()
                        x_snd[s].start()

        for s in range(N_Z):
            x_rcv[s].wait_recv()

        for s in range(N_Z):
            @pl.when(my_z == s)
            def _():
                for tt in range(N_Z):
                    if tt != s:
                        rs[(s, tt)].wait_send()
                        ag[(s, tt)].wait_send()
        for s in range(N_Z):
            x_snd[s].wait_send()

    return pl.pallas_call(
        body,
        out_shape=jax.ShapeDtypeStruct((t, d), partial.dtype),
        in_specs=[pl.BlockSpec(memory_space=pltpu.VMEM)],
        out_specs=pl.BlockSpec(memory_space=pltpu.VMEM),
        scratch_shapes=[
            pltpu.VMEM((N_Z, g, d), partial.dtype),
            pltpu.SemaphoreType.DMA((N_Z,)),
            pltpu.SemaphoreType.DMA((N_Z,)),
            pltpu.SemaphoreType.DMA((N_Z,)),
            pltpu.SemaphoreType.DMA((N_Z,)),
            pltpu.SemaphoreType.DMA((N_Z,)),
            pltpu.SemaphoreType.DMA((N_Z,)),
        ],
        compiler_params=pltpu.CompilerParams(collective_id=0),
    )(partial)


def kernel(ids, E):
    v_per, _ = E.shape
    z = lax.axis_index("z")
    local = ids - z * v_per
    mask = (local >= 0) & (local < v_per)
    safe = jnp.where(mask, local, 0)
    partial = jnp.where(mask[:, None], jnp.take(E, safe, axis=0), 0.0)
    return _allreduce_z(partial.astype(jnp.float32))


# device time: 24598 ns/iter; 2.5209x vs baseline; 1.0100x over previous
import jax
import jax.numpy as jnp
from jax import lax
from jax.experimental import pallas as pl
from jax.experimental.pallas import tpu as pltpu

N_Z = 4
N_W = 2


def _allreduce_z(partial):
    t, d = partial.shape
    half = t // 2
    g = half // N_Z
    wrows = g // N_W

    def body(p_ref, out_ref, rs_buf, rs_send, rs_recv, ag_send, ag_recv,
             x_send, x_recv):
        my_x = lax.axis_index("x")
        my_y = lax.axis_index("y")
        my_z = lax.axis_index("z")
        base = my_x * half
        obase = (1 - my_x) * half

        def wo(b, s, w):
            return b + s * g + w * wrows

        barrier_sem = pltpu.get_barrier_semaphore()
        for nbr in range(N_Z):
            @pl.when(my_z != nbr)
            def _():
                pl.semaphore_signal(
                    barrier_sem, inc=1,
                    device_id=(my_x, my_y, nbr),
                    device_id_type=pl.DeviceIdType.MESH,
                )
        pl.semaphore_signal(
            barrier_sem, inc=1,
            device_id=(1 - my_x, my_y, my_z),
            device_id_type=pl.DeviceIdType.MESH,
        )
        pl.semaphore_wait(barrier_sem, N_Z)

        rs = {}
        ag = {}
        x_snd = {}
        x_rcv = {}
        for w in range(N_W):
            for s in range(N_Z):
                x_snd[(w, s)] = pltpu.make_async_remote_copy(
                    src_ref=out_ref.at[pl.ds(wo(base, s, w), wrows), :],
                    dst_ref=out_ref.at[pl.ds(wo(base, s, w), wrows), :],
                    send_sem=x_send.at[w, s],
                    recv_sem=x_recv.at[w, s],
                    device_id=(1 - my_x, my_y, my_z),
                    device_id_type=pl.DeviceIdType.MESH,
                )
                x_rcv[(w, s)] = pltpu.make_async_remote_copy(
                    src_ref=out_ref.at[pl.ds(wo(obase, s, w), wrows), :],
                    dst_ref=out_ref.at[pl.ds(wo(obase, s, w), wrows), :],
                    send_sem=x_send.at[w, s],
                    recv_sem=x_recv.at[w, s],
                    device_id=(1 - my_x, my_y, my_z),
                    device_id_type=pl.DeviceIdType.MESH,
                )
                for tt in range(N_Z):
                    if tt == s:
                        continue
                    rs[(w, s, tt)] = pltpu.make_async_remote_copy(
                        src_ref=p_ref.at[pl.ds(wo(base, tt, w), wrows), :],
                        dst_ref=rs_buf.at[w, s],
                        send_sem=rs_send.at[w, tt],
                        recv_sem=rs_recv.at[w, s],
                        device_id=(my_x, my_y, tt),
                        device_id_type=pl.DeviceIdType.MESH,
                    )
                    ag[(w, s, tt)] = pltpu.make_async_remote_copy(
                        src_ref=out_ref.at[pl.ds(wo(base, s, w), wrows), :],
                        dst_ref=out_ref.at[pl.ds(wo(base, s, w), wrows), :],
                        send_sem=ag_send.at[w, tt],
                        recv_sem=ag_recv.at[w, s],
                        device_id=(my_x, my_y, tt),
                        device_id_type=pl.DeviceIdType.MESH,
                    )

        for w in range(N_W):
            for s in range(N_Z):
                @pl.when(my_z == s)
                def _():
                    for tt in sorted(
                        (x for x in range(N_Z) if x != s),
                        key=lambda x: abs(x - s),
                    ):
                        rs[(w, s, tt)].start()

        for w in range(N_W):
            for j in range(N_Z):
                @pl.when(my_z == j)
                def _():
                    ro = pl.ds(wo(base, j, w), wrows)
                    order = sorted(
                        (s for s in range(N_Z) if s != j),
                        key=lambda s: abs(s - j),
                    )
                    rs[(w, order[0], j)].wait_recv()
                    out_ref[ro, :] = p_ref[ro, :] + rs_buf[w, order[0], :, :]
                    rs[(w, order[1], j)].wait_recv()
                    out_ref[ro, :] = out_ref[ro, :] + rs_buf[w, order[1], :, :]
                    rs[(w, order[2], j)].wait_recv()
                    out_ref[ro, :] = out_ref[ro, :] + rs_buf[w, order[2], :, :]
                    x_snd[(w, j)].start()
                    for tt in order:
                        ag[(w, j, tt)].start()

        for w in range(N_W):
            for tt in range(N_Z):
                @pl.when(my_z == tt)
                def _():
                    for s in sorted(
                        (x for x in range(N_Z) if x != tt),
                        key=lambda x: abs(x - tt),
                    ):
                        ag[(w, s, tt)].wait_recv()
                        x_snd[(w, s)].start()

        for w in range(N_W):
            for s in range(N_Z):
                x_rcv[(w, s)].wait_recv()

        for w in range(N_W):
            for s in range(N_Z):
                @pl.when(my_z == s)
                def _():
                    for tt in range(N_Z):
                        if tt != s:
                            rs[(w, s, tt)].wait_send()
                            ag[(w, s, tt)].wait_send()
                x_snd[(w, s)].wait_send()

    return pl.pallas_call(
        body,
        out_shape=jax.ShapeDtypeStruct((t, d), partial.dtype),
        in_specs=[pl.BlockSpec(memory_space=pltpu.VMEM)],
        out_specs=pl.BlockSpec(memory_space=pltpu.VMEM),
        scratch_shapes=[
            pltpu.VMEM((N_W, N_Z, wrows, d), partial.dtype),
            pltpu.SemaphoreType.DMA((N_W, N_Z)),
            pltpu.SemaphoreType.DMA((N_W, N_Z)),
            pltpu.SemaphoreType.DMA((N_W, N_Z)),
            pltpu.SemaphoreType.DMA((N_W, N_Z)),
            pltpu.SemaphoreType.DMA((N_W, N_Z)),
            pltpu.SemaphoreType.DMA((N_W, N_Z)),
        ],
        compiler_params=pltpu.CompilerParams(collective_id=0),
    )(partial)


def kernel(ids, E):
    v_per, _ = E.shape
    z = lax.axis_index("z")
    local = ids - z * v_per
    mask = (local >= 0) & (local < v_per)
    safe = jnp.where(mask, local, 0)
    partial = jnp.where(mask[:, None], jnp.take(E, safe, axis=0), 0.0)
    return _allreduce_z(partial.astype(jnp.float32))
